# baseline (speedup 1.0000x reference)
import functools

import jax
import jax.numpy as jnp
import numpy as np
from jax import lax
from jax.experimental import pallas as pl
from jax.experimental.pallas import tpu as pltpu
from jax.experimental.pallas import tpu_sc as plsc

_PAD = 0
_CANDIDATE_MULTIPLE = 2

_NC = 2
_NS = 16
_L = 16

_NEG_INF = np.float32(-np.inf)
_I32_MAX = np.int32(2**31 - 1)
_I32_MIN = np.int32(-(2**31))


_GATHER_DNUMS = lax.GatherDimensionNumbers(
    offset_dims=(), collapsed_slice_dims=(0,), start_index_map=(0,))


def _perm(x, idx):
    return lax.gather(x, idx[:, None], _GATHER_DNUMS, slice_sizes=(1,),
                      mode=lax.GatherScatterMode.PROMISE_IN_BOUNDS)


def _bcast_reduce(x, lanes, op):
    for s in (8, 4, 2, 1):
        x = op(x, _perm(x, lanes ^ s))
    return x


def _insert(x, ix, vs, idxs):
    vs = list(vs)
    idxs = list(idxs)
    for kk in range(len(vs)):
        sel = x > vs[kk]
        nv = jnp.where(sel, x, vs[kk])
        nx = jnp.where(sel, vs[kk], x)
        ni = jnp.where(sel, ix, idxs[kk])
        nix = jnp.where(sel, idxs[kk], ix)
        vs[kk], x = nv, nx
        idxs[kk], ix = ni, nix
    return tuple(vs), tuple(idxs)


def _make_sc_topk(bsz, beam, vocab, k, chunk, group, warm):
    total = beam * vocab
    n_chunks = total // chunk
    n_vregs = chunk // _L
    n_groups = n_vregs // group
    assert total % chunk == 0 and chunk % _L == 0 and n_vregs % group == 0
    assert warm % group == 0 and warm <= n_vregs and n_chunks >= 4
    assert bsz == _NC * _NS

    mesh = plsc.VectorSubcoreMesh(
        core_axis_name="c", subcore_axis_name="s",
        num_cores=_NC, num_subcores=_NS)

    def body(lp_hbm, bias_hbm, mask_hbm, val_hbm, idx_hbm, beam_hbm,
             buf0, buf1, bias_v, mask_v, res_val, res_idx, res_beam,
             vs_ref, ix_ref, th_ref, trig_ref, cnt_ref, sem0, sem1):
        lanes = lax.broadcasted_iota(jnp.int32, (_L,), 0)
        wid = lax.axis_index("s") * _NC + lax.axis_index("c")
        b = wid

        pltpu.sync_copy(bias_hbm.at[pl.ds(b * _L, _L)], bias_v)
        pltpu.sync_copy(mask_hbm.at[pl.ds(b * _L, _L)], mask_v)
        bias_vec = bias_v[...]
        mask_vec = mask_v[...]

        bufs = (buf0, buf1)
        sems = (sem0, sem1)

        def chunk_src(g):
            return lp_hbm.at[pl.ds(b * total + g * chunk, chunk)]

        pltpu.async_copy(chunk_src(0), buf0, sem0)

        def load_lists():
            vs = [vs_ref[pl.ds(kk * _L, _L)] for kk in range(k)]
            idxs = [ix_ref[pl.ds(kk * _L, _L)] for kk in range(k)]
            return vs, idxs

        def store_lists(vs, idxs):
            for kk in range(k):
                vs_ref[pl.ds(kk * _L, _L)] = vs[kk]
                ix_ref[pl.ds(kk * _L, _L)] = idxs[kk]

        def any_lane(pred):
            return plsc.all_reduce_population_count(pred)[0] > 0

        def tree_max(vals):
            while len(vals) > 1:
                nxt = [jnp.maximum(vals[i], vals[i + 1])
                       for i in range(0, len(vals) - 1, 2)]
                if len(vals) % 2:
                    nxt.append(vals[-1])
                vals = nxt
            return vals[0]

        def process(buf, g, start):
            beam_g = (g * chunk) // vocab
            bias_s = _bcast_reduce(
                jnp.where(lanes == beam_g, bias_vec, _NEG_INF),
                lanes, jnp.maximum)
            base = g * chunk

            def thresh(v7):
                t = v7 - bias_s
                return t - (jnp.abs(t) + jnp.abs(v7)) * np.float32(1e-6)

            th_ref[...] = thresh(vs_ref[pl.ds((k - 1) * _L, _L)])
            cnt_ref[0] = jnp.int32(0)

            def group_body(t, _):
                xs = [buf[pl.ds(t * (group * _L) + u * _L, _L)]
                      for u in range(group)]
                gm = tree_max(xs)

                @pl.when(any_lane(gm > th_ref[...]))
                def _():
                    c = cnt_ref[0]
                    trig_ref[c] = t
                    cnt_ref[0] = c + 1

                return 0

            lax.fori_loop(start, n_groups, group_body, 0)

            def handle(i, _):
                t = trig_ref[i]
                xs = [buf[pl.ds(t * (group * _L) + u * _L, _L)]
                      for u in range(group)]
                gbase = base + t * (group * _L)
                ixs = [gbase + u * _L + lanes for u in range(group)]

                def argmax_round(xs_c):
                    pairs = list(zip(xs_c, ixs))
                    while len(pairs) > 1:
                        nxt = []
                        for i2 in range(0, len(pairs) - 1, 2):
                            (av, ai), (bv, bi) = pairs[i2], pairs[i2 + 1]
                            s = bv > av
                            nxt.append((jnp.where(s, bv, av),
                                        jnp.where(s, bi, ai)))
                        if len(pairs) % 2:
                            nxt.append(pairs[-1])
                        pairs = nxt
                    mv, mi = pairs[0]
                    vs, idxs = load_lists()
                    vs, idxs = _insert(mv + bias_s, mi, vs, idxs)
                    store_lists(vs, idxs)
                    th_n = thresh(vs[k - 1])
                    th_ref[...] = th_n
                    xs_n = [jnp.where(ixs[u] == mi, _NEG_INF, xs_c[u])
                            for u in range(group)]
                    return xs_n, th_n

                def hit_any(xs_c, th_c):
                    return any_lane(tree_max(list(xs_c)) > th_c)

                @pl.when(hit_any(xs, th_ref[...]))
                def _():
                    xs2, th2 = argmax_round(xs)

                    @pl.when(hit_any(xs2, th2))
                    def _():
                        lax.while_loop(
                            lambda c: hit_any(c[0], c[1]),
                            lambda c: (lambda r: (tuple(r[0]), r[1]))(
                                argmax_round(list(c[0]))),
                            (tuple(xs2), th2))

                return 0

            lax.fori_loop(0, cnt_ref[0], handle, 0)

        pltpu.make_async_copy(chunk_src(0), buf0, sem0).wait()
        pltpu.async_copy(chunk_src(1), buf1, sem1)
        bias0 = _bcast_reduce(jnp.where(lanes == 0, bias_vec, _NEG_INF),
                              lanes, jnp.maximum)

        def warm_body(i, c):
            vs, idxs = c
            x = buf0[pl.ds(i * _L, _L)] + bias0
            return _insert(x, i * _L + lanes, vs, idxs)

        vs0, idxs0 = lax.fori_loop(
            0, warm, warm_body,
            (tuple(jnp.full((_L,), _NEG_INF) for _ in range(k)),
             tuple(jnp.zeros((_L,), jnp.int32) for _ in range(k))))
        store_lists(list(vs0), list(idxs0))
        process(buf0, 0, warm // group)

        def chunk_step(g, _):
            def do(buf, sem, obuf, osem):
                pltpu.make_async_copy(chunk_src(0), buf, sem).wait()

                @pl.when(g + 1 < n_chunks)
                def _():
                    pltpu.async_copy(chunk_src(g + 1), obuf, osem)

                process(buf, g, 0)
                return 0

            return lax.cond(g % 2 == 0,
                            lambda: do(buf0, sem0, buf1, sem1),
                            lambda: do(buf1, sem1, buf0, sem0))

        lax.fori_loop(1, n_chunks, chunk_step, 0)

        vs, idxs = load_lists()
        rv = jnp.full((_L,), _NEG_INF)
        ri = jnp.zeros((_L,), jnp.int32)
        rb = jnp.zeros((_L,), jnp.int32)
        for j in range(k):
            gmax = _bcast_reduce(vs[0], lanes, jnp.maximum)
            gidx = _bcast_reduce(
                jnp.where(vs[0] == gmax, idxs[0], _I32_MAX),
                lanes, jnp.minimum)
            pop = (vs[0] == gmax) & (idxs[0] == gidx)
            beam_s = gidx // vocab
            vocab_s = gidx % vocab
            mval = _bcast_reduce(
                jnp.where(lanes == beam_s, mask_vec, _I32_MIN),
                lanes, jnp.maximum)
            vocab_s = jnp.where(mval == 0, np.int32(_PAD), vocab_s)
            rv = jnp.where(lanes == j, gmax, rv)
            ri = jnp.where(lanes == j, vocab_s, ri)
            rb = jnp.where(lanes == j, beam_s, rb)
            for kk in range(k - 1):
                vs[kk] = jnp.where(pop, vs[kk + 1], vs[kk])
                idxs[kk] = jnp.where(pop, idxs[kk + 1], idxs[kk])
            vs[k - 1] = jnp.where(pop, _NEG_INF, vs[k - 1])
            idxs[k - 1] = jnp.where(pop, np.int32(0), idxs[k - 1])

        res_val[...] = rv
        res_idx[...] = ri
        res_beam[...] = rb
        pltpu.sync_copy(res_val, val_hbm.at[pl.ds(b * _L, _L)])
        pltpu.sync_copy(res_idx, idx_hbm.at[pl.ds(b * _L, _L)])
        pltpu.sync_copy(res_beam, beam_hbm.at[pl.ds(b * _L, _L)])

    return pl.kernel(
        body,
        out_type=(
            jax.ShapeDtypeStruct((bsz * _L,), jnp.float32),
            jax.ShapeDtypeStruct((bsz * _L,), jnp.int32),
            jax.ShapeDtypeStruct((bsz * _L,), jnp.int32),
        ),
        mesh=mesh,
        compiler_params=pltpu.CompilerParams(
            needs_layout_passes=False,
            disable_bounds_checks=True,
            disable_semaphore_checks=True,
        ),
        scratch_types=[
            pltpu.VMEM((chunk,), jnp.float32),
            pltpu.VMEM((chunk,), jnp.float32),
            pltpu.VMEM((_L,), jnp.float32),
            pltpu.VMEM((_L,), jnp.int32),
            pltpu.VMEM((_L,), jnp.float32),
            pltpu.VMEM((_L,), jnp.int32),
            pltpu.VMEM((_L,), jnp.int32),
            pltpu.VMEM((k * _L,), jnp.float32),
            pltpu.VMEM((k * _L,), jnp.int32),
            pltpu.VMEM((_L,), jnp.float32),
            pltpu.SMEM((chunk // _L // group,), jnp.int32),
            pltpu.SMEM((1,), jnp.int32),
            pltpu.SemaphoreType.DMA,
            pltpu.SemaphoreType.DMA,
        ],
    )


def kernel(step, lprobs, scores, mask_stop_search):
    bsz, beam, vocab = lprobs.shape
    k = _CANDIDATE_MULTIPLE * beam

    step = jnp.asarray(step, jnp.int32)
    bias_later = jnp.take(scores, step, axis=2)
    bias_step0 = jnp.where(jnp.arange(beam) == 0, 0.0, -jnp.inf)[None, :]
    bias = jnp.where(step == 0, bias_step0, bias_later).astype(jnp.float32)

    bias_p = jnp.pad(bias, ((0, 0), (0, _L - beam)),
                     constant_values=-jnp.inf).reshape(-1)
    mask_p = jnp.pad(mask_stop_search.astype(jnp.int32),
                     ((0, 0), (0, _L - beam))).reshape(-1)
    lp_flat = lprobs.reshape(-1)

    topk = _make_sc_topk(bsz, beam, vocab, k, chunk=20000, group=50, warm=150)
    vals, inds, beams = topk(lp_flat, bias_p, mask_p)
    return (vals.reshape(bsz, _L)[:, :k],
            inds.reshape(bsz, _L)[:, :k],
            beams.reshape(bsz, _L)[:, :k])


# speedup vs baseline: 1.3798x; 1.3798x over previous
import functools

import jax
import jax.numpy as jnp
import numpy as np
from jax import lax
from jax.experimental import pallas as pl
from jax.experimental.pallas import tpu as pltpu
from jax.experimental.pallas import tpu_sc as plsc

_PAD = 0
_CANDIDATE_MULTIPLE = 2

_NC = 2
_NS = 16
_L = 16

_NEG_INF = np.float32(-np.inf)
_I32_MAX = np.int32(2**31 - 1)
_I32_MIN = np.int32(-(2**31))


_GATHER_DNUMS = lax.GatherDimensionNumbers(
    offset_dims=(), collapsed_slice_dims=(0,), start_index_map=(0,))


def _perm(x, idx):
    return lax.gather(x, idx[:, None], _GATHER_DNUMS, slice_sizes=(1,),
                      mode=lax.GatherScatterMode.PROMISE_IN_BOUNDS)


def _bcast_reduce(x, lanes, op):
    for s in (8, 4, 2, 1):
        x = op(x, _perm(x, lanes ^ s))
    return x


def _insert(x, ix, vs, idxs):
    vs = list(vs)
    idxs = list(idxs)
    for kk in range(len(vs)):
        sel = x > vs[kk]
        nv = jnp.where(sel, x, vs[kk])
        nx = jnp.where(sel, vs[kk], x)
        ni = jnp.where(sel, ix, idxs[kk])
        nix = jnp.where(sel, idxs[kk], ix)
        vs[kk], x = nv, nx
        idxs[kk], ix = ni, nix
    return tuple(vs), tuple(idxs)


def _make_sc_topk(bsz, beam, vocab, k, chunk, group, warm):
    total = beam * vocab
    n_chunks = total // chunk
    n_vregs = chunk // _L
    n_groups = n_vregs // group
    assert total % chunk == 0 and chunk % _L == 0 and n_vregs % group == 0
    assert warm % group == 0 and warm <= n_vregs and n_chunks >= 4
    assert bsz == _NC * _NS

    mesh = plsc.VectorSubcoreMesh(
        core_axis_name="c", subcore_axis_name="s",
        num_cores=_NC, num_subcores=_NS)

    def body(lp_hbm, bias_hbm, mask_hbm, val_hbm, idx_hbm, beam_hbm,
             buf0, buf1, bias_v, mask_v, res_val, res_idx, res_beam,
             vs_ref, ix_ref, th_ref, trig_ref, cnt_ref, sem0, sem1):
        lanes = lax.broadcasted_iota(jnp.int32, (_L,), 0)
        wid = lax.axis_index("s") * _NC + lax.axis_index("c")
        b = wid

        pltpu.sync_copy(bias_hbm.at[pl.ds(b * _L, _L)], bias_v)
        pltpu.sync_copy(mask_hbm.at[pl.ds(b * _L, _L)], mask_v)
        bias_vec = bias_v[...]
        mask_vec = mask_v[...]

        bufs = (buf0, buf1)
        sems = (sem0, sem1)

        def chunk_src(g):
            return lp_hbm.at[pl.ds(b * total + g * chunk, chunk)]

        pltpu.async_copy(chunk_src(0), buf0, sem0)

        def load_lists():
            vs = [vs_ref[pl.ds(kk * _L, _L)] for kk in range(k)]
            idxs = [ix_ref[pl.ds(kk * _L, _L)] for kk in range(k)]
            return vs, idxs

        def store_lists(vs, idxs):
            for kk in range(k):
                vs_ref[pl.ds(kk * _L, _L)] = vs[kk]
                ix_ref[pl.ds(kk * _L, _L)] = idxs[kk]

        def any_lane(pred):
            return plsc.all_reduce_population_count(pred)[0] > 0

        def tree_max(vals):
            while len(vals) > 1:
                nxt = [jnp.maximum(vals[i], vals[i + 1])
                       for i in range(0, len(vals) - 1, 2)]
                if len(vals) % 2:
                    nxt.append(vals[-1])
                vals = nxt
            return vals[0]

        def process(buf, g, start):
            beam_g = (g * chunk) // vocab
            bias_s = _bcast_reduce(
                jnp.where(lanes == beam_g, bias_vec, _NEG_INF),
                lanes, jnp.maximum)
            base = g * chunk

            def thresh(v7):
                t = v7 - bias_s
                return t - (jnp.abs(t) + jnp.abs(v7)) * np.float32(1e-6)

            th_ref[...] = thresh(vs_ref[pl.ds((k - 1) * _L, _L)])
            cnt_ref[0] = jnp.int32(0)

            def group_body(t, _):
                xs = [buf[pl.ds(t * (group * _L) + u * _L, _L)]
                      for u in range(group)]
                gm = tree_max(xs)

                @pl.when(any_lane(gm > th_ref[...]))
                def _():
                    c = cnt_ref[0]
                    trig_ref[c] = t
                    cnt_ref[0] = c + 1

                return 0

            lax.fori_loop(start, n_groups, group_body, 0)

            def handle(i, _):
                t = trig_ref[i]
                xs = [buf[pl.ds(t * (group * _L) + u * _L, _L)]
                      for u in range(group)]
                gbase = base + t * (group * _L)
                ixs = [gbase + u * _L + lanes for u in range(group)]

                def argmax_round(xs_c):
                    pairs = list(zip(xs_c, ixs))
                    while len(pairs) > 1:
                        nxt = []
                        for i2 in range(0, len(pairs) - 1, 2):
                            (av, ai), (bv, bi) = pairs[i2], pairs[i2 + 1]
                            s = bv > av
                            nxt.append((jnp.where(s, bv, av),
                                        jnp.where(s, bi, ai)))
                        if len(pairs) % 2:
                            nxt.append(pairs[-1])
                        pairs = nxt
                    mv, mi = pairs[0]
                    vs, idxs = load_lists()
                    vs, idxs = _insert(mv + bias_s, mi, vs, idxs)
                    store_lists(vs, idxs)
                    th_n = thresh(vs[k - 1])
                    th_ref[...] = th_n
                    xs_n = [jnp.where(ixs[u] == mi, _NEG_INF, xs_c[u])
                            for u in range(group)]
                    return xs_n, th_n

                def hit_any(xs_c, th_c):
                    return any_lane(tree_max(list(xs_c)) > th_c)

                @pl.when(hit_any(xs, th_ref[...]))
                def _():
                    xs2, th2 = argmax_round(xs)

                    @pl.when(hit_any(xs2, th2))
                    def _():
                        lax.while_loop(
                            lambda c: hit_any(c[0], c[1]),
                            lambda c: (lambda r: (tuple(r[0]), r[1]))(
                                argmax_round(list(c[0]))),
                            (tuple(xs2), th2))

                return 0

            lax.fori_loop(0, cnt_ref[0], handle, 0)

        pltpu.make_async_copy(chunk_src(0), buf0, sem0).wait()
        pltpu.async_copy(chunk_src(1), buf1, sem1)
        bias0 = _bcast_reduce(jnp.where(lanes == 0, bias_vec, _NEG_INF),
                              lanes, jnp.maximum)

        def warm_body(i, c):
            vs, idxs = c
            x = buf0[pl.ds(i * _L, _L)] + bias0
            return _insert(x, i * _L + lanes, vs, idxs)

        vs0, idxs0 = lax.fori_loop(
            0, warm, warm_body,
            (tuple(jnp.full((_L,), _NEG_INF) for _ in range(k)),
             tuple(jnp.zeros((_L,), jnp.int32) for _ in range(k))))
        store_lists(list(vs0), list(idxs0))
        process(buf0, 0, warm // group)

        def chunk_step(g, _):
            def do(buf, sem, obuf, osem):
                pltpu.make_async_copy(chunk_src(0), buf, sem).wait()

                @pl.when(g + 1 < n_chunks)
                def _():
                    pltpu.async_copy(chunk_src(g + 1), obuf, osem)

                process(buf, g, 0)
                return 0

            return lax.cond(g % 2 == 0,
                            lambda: do(buf0, sem0, buf1, sem1),
                            lambda: do(buf1, sem1, buf0, sem0))

        lax.fori_loop(1, n_chunks, chunk_step, 0)

        vs, idxs = load_lists()
        rv = jnp.full((_L,), _NEG_INF)
        ri = jnp.zeros((_L,), jnp.int32)
        rb = jnp.zeros((_L,), jnp.int32)
        for j in range(k):
            gmax = _bcast_reduce(vs[0], lanes, jnp.maximum)
            gidx = _bcast_reduce(
                jnp.where(vs[0] == gmax, idxs[0], _I32_MAX),
                lanes, jnp.minimum)
            pop = (vs[0] == gmax) & (idxs[0] == gidx)
            beam_s = gidx // vocab
            vocab_s = gidx % vocab
            mval = _bcast_reduce(
                jnp.where(lanes == beam_s, mask_vec, _I32_MIN),
                lanes, jnp.maximum)
            vocab_s = jnp.where(mval == 0, np.int32(_PAD), vocab_s)
            rv = jnp.where(lanes == j, gmax, rv)
            ri = jnp.where(lanes == j, vocab_s, ri)
            rb = jnp.where(lanes == j, beam_s, rb)
            for kk in range(k - 1):
                vs[kk] = jnp.where(pop, vs[kk + 1], vs[kk])
                idxs[kk] = jnp.where(pop, idxs[kk + 1], idxs[kk])
            vs[k - 1] = jnp.where(pop, _NEG_INF, vs[k - 1])
            idxs[k - 1] = jnp.where(pop, np.int32(0), idxs[k - 1])

        res_val[...] = rv
        res_idx[...] = ri
        res_beam[...] = rb
        pltpu.sync_copy(res_val, val_hbm.at[pl.ds(b * _L, _L)])
        pltpu.sync_copy(res_idx, idx_hbm.at[pl.ds(b * _L, _L)])
        pltpu.sync_copy(res_beam, beam_hbm.at[pl.ds(b * _L, _L)])

    return pl.kernel(
        body,
        out_type=(
            jax.ShapeDtypeStruct((bsz * _L,), jnp.float32),
            jax.ShapeDtypeStruct((bsz * _L,), jnp.int32),
            jax.ShapeDtypeStruct((bsz * _L,), jnp.int32),
        ),
        mesh=mesh,
        compiler_params=pltpu.CompilerParams(
            needs_layout_passes=False,
            disable_bounds_checks=True,
            disable_semaphore_checks=True,
        ),
        scratch_types=[
            pltpu.VMEM((chunk,), jnp.float32),
            pltpu.VMEM((chunk,), jnp.float32),
            pltpu.VMEM((_L,), jnp.float32),
            pltpu.VMEM((_L,), jnp.int32),
            pltpu.VMEM((_L,), jnp.float32),
            pltpu.VMEM((_L,), jnp.int32),
            pltpu.VMEM((_L,), jnp.int32),
            pltpu.VMEM((k * _L,), jnp.float32),
            pltpu.VMEM((k * _L,), jnp.int32),
            pltpu.VMEM((_L,), jnp.float32),
            pltpu.SMEM((chunk // _L // group,), jnp.int32),
            pltpu.SMEM((1,), jnp.int32),
            pltpu.SemaphoreType.DMA,
            pltpu.SemaphoreType.DMA,
        ],
    )


def kernel(step, lprobs, scores, mask_stop_search):
    bsz, beam, vocab = lprobs.shape
    k = _CANDIDATE_MULTIPLE * beam

    step = jnp.asarray(step, jnp.int32)
    bias_later = jnp.take(scores, step, axis=2)
    bias_step0 = jnp.where(jnp.arange(beam) == 0, 0.0, -jnp.inf)[None, :]
    bias = jnp.where(step == 0, bias_step0, bias_later).astype(jnp.float32)

    bias_p = jnp.pad(bias, ((0, 0), (0, _L - beam)),
                     constant_values=-jnp.inf).reshape(-1)
    mask_p = jnp.pad(mask_stop_search.astype(jnp.int32),
                     ((0, 0), (0, _L - beam))).reshape(-1)
    lp_flat = lprobs.reshape(-1)

    topk = _make_sc_topk(bsz, beam, vocab, k, chunk=20000, group=25, warm=375)
    vals, inds, beams = topk(lp_flat, bias_p, mask_p)
    return (vals.reshape(bsz, _L)[:, :k],
            inds.reshape(bsz, _L)[:, :k],
            beams.reshape(bsz, _L)[:, :k])
